# baseline (speedup 1.0000x reference)
import functools

import jax
import jax.numpy as jnp
from jax import lax
from jax.experimental import pallas as pl
from jax.experimental.pallas import tpu as pltpu
from jax.experimental.pallas import tpu_sc as plsc

N = 10000
E = 160000
D = 256
DH = 128
NSC = 16
N_PAD = 10240
R = N_PAD // NSC
EPT = E // NSC
B = 80
NCHUNK = EPT // B

BM = 400


def _pack_perms():
    import numpy as _np
    lo, hi = [], []
    for h in range(2):
        for j in range(DH // 32):
            for t in range(16):
                lo.append(128 * h + 32 * j + t)
                hi.append(128 * h + 32 * j + 16 + t)
    return _np.asarray(lo), _np.asarray(hi)


_PERM_LO, _PERM_HI = _pack_perms()


def _pack_i32(ylo, yhi):
    lo16 = jax.lax.bitcast_convert_type(ylo.astype(jnp.bfloat16), jnp.uint16)
    hi16 = jax.lax.bitcast_convert_type(yhi.astype(jnp.bfloat16), jnp.uint16)
    w = lo16.astype(jnp.uint32) | (hi16.astype(jnp.uint32) << 16)
    return jax.lax.bitcast_convert_type(w, jnp.int32)


def _mm_y_body(x_ref, wlo_ref, whi_ref, ya_ref, yb_ref):
    xb = x_ref[...]
    ylo = jnp.dot(xb, wlo_ref[...], preferred_element_type=jnp.float32)
    yhi = jnp.dot(xb, whi_ref[...], preferred_element_type=jnp.float32)
    w = _pack_i32(ylo, yhi)
    ya_ref[...] = w[:, :DH // 2]
    yb_ref[...] = w[:, DH // 2:]


def _mm_y(x, wlo, whi):
    return pl.pallas_call(
        _mm_y_body,
        grid=(N // BM,),
        in_specs=[
            pl.BlockSpec((BM, D), lambda i: (i, 0)),
            pl.BlockSpec((D, D // 2), lambda i: (0, 0)),
            pl.BlockSpec((D, D // 2), lambda i: (0, 0)),
        ],
        out_specs=[
            pl.BlockSpec((BM, DH // 2), lambda i: (i, 0)),
            pl.BlockSpec((BM, DH // 2), lambda i: (i, 0)),
        ],
        out_shape=[
            jax.ShapeDtypeStruct((N_PAD, DH // 2), jnp.int32),
            jax.ShapeDtypeStruct((N_PAD, DH // 2), jnp.int32),
        ],
    )(x, wlo, whi)


def _mm_s_body(x_ref, ws_ref, b_ref, s_ref):
    s_ref[...] = jnp.dot(x_ref[...], ws_ref[...], preferred_element_type=jnp.float32) + b_ref[...]


def _mm_s(x, ws, b):
    return pl.pallas_call(
        _mm_s_body,
        grid=(N // BM,),
        in_specs=[
            pl.BlockSpec((BM, D), lambda i: (i, 0)),
            pl.BlockSpec((D, D), lambda i: (0, 0)),
            pl.BlockSpec((1, D), lambda i: (0, 0)),
        ],
        out_specs=pl.BlockSpec((BM, D), lambda i: (i, 0)),
        out_shape=jax.ShapeDtypeStruct((N_PAD, D), jnp.float32),
    )(x, ws, b.reshape(1, D))


def _h_of(s1_ref, aa_ref, ab_ref, c0_ref, c1_ref):
    inv = 1.0 / jnp.maximum(c0_ref[...] + c1_ref[...], 1.0)
    agg = jnp.concatenate([aa_ref[...], ab_ref[...]], axis=1) * inv
    return jnp.maximum(s1_ref[...] + agg, 0.0)


def _mid_y_body(s1_ref, aa_ref, ab_ref, c0_ref, c1_ref, wlo_ref, whi_ref,
                ya_ref, yb_ref):
    h = _h_of(s1_ref, aa_ref, ab_ref, c0_ref, c1_ref)
    ylo = jnp.dot(h, wlo_ref[...], preferred_element_type=jnp.float32)
    yhi = jnp.dot(h, whi_ref[...], preferred_element_type=jnp.float32)
    w = _pack_i32(ylo, yhi)
    ya_ref[...] = w[:, :DH // 2]
    yb_ref[...] = w[:, DH // 2:]


def _mid_s_body(s1_ref, aa_ref, ab_ref, c0_ref, c1_ref, ws_ref, b_ref, s2_ref):
    h = _h_of(s1_ref, aa_ref, ab_ref, c0_ref, c1_ref)
    s2_ref[...] = jnp.dot(h, ws_ref[...], preferred_element_type=jnp.float32) + b_ref[...]


_MID_IN = [
    pl.BlockSpec((BM, D), lambda i: (i, 0)),
    pl.BlockSpec((BM, DH), lambda i: (i, 0)),
    pl.BlockSpec((BM, DH), lambda i: (i, 0)),
    pl.BlockSpec((BM, 1), lambda i: (i, 0)),
    pl.BlockSpec((BM, 1), lambda i: (i, 0)),
    pl.BlockSpec((D, D // 2), lambda i: (0, 0)),
]


def _mid_y(s1, aa, ab, c0, c1, wlo, whi):
    return pl.pallas_call(
        _mid_y_body,
        grid=(N // BM,),
        in_specs=_MID_IN + [pl.BlockSpec((D, D // 2), lambda i: (0, 0))],
        out_specs=[
            pl.BlockSpec((BM, DH // 2), lambda i: (i, 0)),
            pl.BlockSpec((BM, DH // 2), lambda i: (i, 0)),
        ],
        out_shape=[
            jax.ShapeDtypeStruct((N_PAD, DH // 2), jnp.int32),
            jax.ShapeDtypeStruct((N_PAD, DH // 2), jnp.int32),
        ],
    )(s1, aa, ab, c0, c1, wlo, whi)


def _mid_s(s1, aa, ab, c0, c1, ws, b):
    return pl.pallas_call(
        _mid_s_body,
        grid=(N // BM,),
        in_specs=_MID_IN[:5] + [pl.BlockSpec((D, D), lambda i: (0, 0)),
                                pl.BlockSpec((1, D), lambda i: (0, 0))],
        out_specs=pl.BlockSpec((BM, D), lambda i: (i, 0)),
        out_shape=jax.ShapeDtypeStruct((N_PAD, D), jnp.float32),
    )(s1, aa, ab, c0, c1, ws, b.reshape(1, D))


def _fin_body(s2_ref, aa_ref, ab_ref, c0_ref, c1_ref, o_ref):
    inv = 1.0 / jnp.maximum(c0_ref[...] + c1_ref[...], 1.0)
    o_ref[...] = s2_ref[...] + jnp.concatenate([aa_ref[...], ab_ref[...]], axis=1) * inv


def _fin(s2, aa, ab, c0, c1):
    return pl.pallas_call(
        _fin_body,
        grid=(N // BM,),
        in_specs=[
            pl.BlockSpec((BM, D), lambda i: (i, 0)),
            pl.BlockSpec((BM, DH), lambda i: (i, 0)),
            pl.BlockSpec((BM, DH), lambda i: (i, 0)),
            pl.BlockSpec((BM, 1), lambda i: (i, 0)),
            pl.BlockSpec((BM, 1), lambda i: (i, 0)),
        ],
        out_specs=pl.BlockSpec((BM, D), lambda i: (i, 0)),
        out_shape=jax.ShapeDtypeStruct((N, D), jnp.float32),
    )(s2, aa, ab, c0, c1)


ECNT = E // 32


@functools.lru_cache(maxsize=None)
def _make_segsum():
    mesh = plsc.VectorSubcoreMesh(
        core_axis_name="c", subcore_axis_name="s", num_cores=2, num_subcores=NSC)

    DP = 4
    IP = 6

    out_type = [
        jax.ShapeDtypeStruct((N_PAD, DH), jnp.float32),
        jax.ShapeDtypeStruct((N_PAD, DH), jnp.float32),
    ]
    scratch = [
        pltpu.VMEM((IP, B), jnp.int32),
        pltpu.VMEM((IP, B), jnp.int32),
        pltpu.VMEM((DP, B, DH // 2), jnp.int32),
        pltpu.VMEM((2, B, DH), jnp.float32),
        pltpu.VMEM_SHARED((N_PAD, DH), jnp.float32),
        pltpu.SemaphoreType.DMA,
        pltpu.SemaphoreType.DMA,
        pltpu.SemaphoreType.DMA,
        pltpu.SemaphoreType.DMA,
    ]

    def body(ya, yb, ei, agg_a, agg_b,
             sidx, didx, rows, rowsf, accum, sisem, disem, gsem, ssem):
        c = lax.axis_index("c")
        s = lax.axis_index("s")
        row0 = s * R
        ebase = s * EPT

        def _zr(r, carry):
            for k in range(DH // 16):
                rowsf[0, r, pl.ds(k * 16, 16)] = jnp.zeros((16,), jnp.float32)
            return carry
        lax.fori_loop(0, B, _zr, 0)
        for q in range(R // B):
            pltpu.sync_copy(rowsf.at[0], accum.at[pl.ds(row0 + q * B, B)])
        plsc.subcore_barrier()

        def load_idx(i):
            buf = lax.rem(i, IP)
            off = ebase + i * B
            pltpu.async_copy(ei.at[pl.ds(off, B)], sidx.at[buf], sisem)
            pltpu.async_copy(ei.at[pl.ds(E + off, B)], didx.at[buf], disem)

        def gather(i):
            buf = lax.rem(i, DP)

            @pl.when(c == 0)
            def _():
                pltpu.async_copy(ya.at[sidx.at[lax.rem(i, IP)]], rows.at[buf], gsem)

            @pl.when(c == 1)
            def _():
                pltpu.async_copy(yb.at[sidx.at[lax.rem(i, IP)]], rows.at[buf], gsem)

        def wait_idx(sem):
            pltpu.make_async_copy(ei.at[pl.ds(0, B)], sidx.at[0], sem).wait()

        def wait_gather():
            pltpu.make_async_copy(ya.at[sidx.at[0]], rows.at[0], gsem).wait()

        def wait_scatter():
            pltpu.make_async_copy(rowsf.at[0], accum.at[didx.at[0]], ssem).wait()

        for k in range(4):
            load_idx(k)
        wait_idx(sisem)
        gather(0)
        wait_idx(sisem)
        gather(1)

        def step(i, carry):
            @pl.when(i >= 2)
            def _():
                wait_scatter()

            @pl.when(i + 4 < NCHUNK)
            def _():
                load_idx(i + 4)

            @pl.when(i + 2 < NCHUNK)
            def _():
                wait_idx(sisem)
                gather(i + 2)

            wait_gather()
            gbuf = lax.rem(i, DP)
            fbuf = lax.rem(i, 2)
            mask_hi = jnp.full((16,), -65536, jnp.int32)

            def _cv(r, carry):
                for g in range(DH // 32):
                    p = rows[gbuf, r, pl.ds(g * 16, 16)]
                    lo = plsc.bitcast(p << 16, jnp.float32)
                    hi = plsc.bitcast(p & mask_hi, jnp.float32)
                    rowsf[fbuf, r, pl.ds(g * 32, 16)] = lo
                    rowsf[fbuf, r, pl.ds(g * 32 + 16, 16)] = hi
                return carry
            lax.fori_loop(0, B, _cv, 0)
            wait_idx(disem)
            pltpu.async_copy(rowsf.at[fbuf],
                             accum.at[didx.at[lax.rem(i, IP)]], ssem, add=True)
            return carry

        lax.fori_loop(0, NCHUNK, step, 0)
        wait_scatter()
        wait_scatter()
        plsc.subcore_barrier()

        @pl.when(c == 0)
        def _():
            pltpu.sync_copy(accum.at[pl.ds(row0, R)], agg_a.at[pl.ds(row0, R)])

        @pl.when(c == 1)
        def _():
            pltpu.sync_copy(accum.at[pl.ds(row0, R)], agg_b.at[pl.ds(row0, R)])

    return pl.kernel(
        body, out_type=out_type, mesh=mesh, scratch_types=scratch,
        compiler_params=pltpu.CompilerParams(
            needs_layout_passes=False, use_tc_tiling_on_sc=False))


@functools.lru_cache(maxsize=None)
def _make_cnt():
    mesh = plsc.VectorSubcoreMesh(
        core_axis_name="c", subcore_axis_name="s", num_cores=2, num_subcores=NSC)

    out_type = [
        jax.ShapeDtypeStruct((N_PAD,), jnp.float32),
        jax.ShapeDtypeStruct((N_PAD,), jnp.float32),
    ]
    scratch = [
        pltpu.VMEM((ECNT,), jnp.int32),
        pltpu.VMEM((N_PAD,), jnp.float32),
        pltpu.VMEM((NSC, R), jnp.float32),
        pltpu.VMEM((R,), jnp.float32),
        pltpu.VMEM_SHARED((NSC, N_PAD), jnp.float32),
    ]

    def body(ei, c0_out, c1_out, didx, cntp, cred, cout, cnt_all):
        c = lax.axis_index("c")
        s = lax.axis_index("s")
        w = c * NSC + s
        row0 = s * R

        pltpu.sync_copy(ei.at[pl.ds(E + w * ECNT, ECNT)], didx)

        def _zc(i, carry):
            cntp[pl.ds(i * 16, 16)] = jnp.zeros((16,), jnp.float32)
            return carry
        lax.fori_loop(0, N_PAD // 16, _zc, 0)

        ones16 = jnp.ones((16,), jnp.float32)

        def _cc(i, carry):
            d16 = didx[pl.ds(i * 16, 16)]
            plsc.addupdate_scatter(cntp, [d16], ones16)
            return carry
        lax.fori_loop(0, ECNT // 16, _cc, 0)
        d16 = didx[pl.ds(ECNT - 16, 16)]
        lanes = lax.broadcasted_iota(jnp.int32, (16,), 0)
        plsc.addupdate_scatter(cntp, [d16], ones16, mask=lanes >= 8)

        pltpu.sync_copy(cntp, cnt_all.at[s])
        plsc.subcore_barrier()
        pltpu.sync_copy(cnt_all.at[:, pl.ds(row0, R)], cred)

        def red(j, carry):
            acc = jnp.zeros((16,), jnp.float32)
            for r in range(NSC):
                acc = acc + cred[r, pl.ds(j * 16, 16)]
            cout[pl.ds(j * 16, 16)] = acc
            return carry
        lax.fori_loop(0, R // 16, red, 0)

        @pl.when(c == 0)
        def _():
            pltpu.sync_copy(cout, c0_out.at[pl.ds(row0, R)])

        @pl.when(c == 1)
        def _():
            pltpu.sync_copy(cout, c1_out.at[pl.ds(row0, R)])

    return pl.kernel(
        body, out_type=out_type, mesh=mesh, scratch_types=scratch,
        compiler_params=pltpu.CompilerParams(needs_layout_passes=False))


@jax.jit
def kernel(x, edge_index, W1_self, W1_neigh, b1, W2_self, W2_neigh, b2):
    ei = edge_index.reshape(2 * E)
    cnt0, cnt1 = _make_cnt()(ei)
    c0 = cnt0.reshape(N_PAD, 1)
    c1 = cnt1.reshape(N_PAD, 1)
    y1a, y1b = _mm_y(x, W1_neigh[:, _PERM_LO], W1_neigh[:, _PERM_HI])
    agg_a, agg_b = _make_segsum()(y1a, y1b, ei)
    s1 = _mm_s(x, W1_self, b1)
    y2a, y2b = _mid_y(s1, agg_a, agg_b, c0, c1,
                      W2_neigh[:, _PERM_LO], W2_neigh[:, _PERM_HI])
    agg_a2, agg_b2 = _make_segsum()(y2a, y2b, ei)
    s2 = _mid_s(s1, agg_a, agg_b, c0, c1, W2_self, b2)
    return _fin(s2, agg_a2, agg_b2, c0, c1)


# speedup vs baseline: 2.0442x; 2.0442x over previous
import functools

import jax
import jax.numpy as jnp
from jax import lax
from jax.experimental import pallas as pl
from jax.experimental.pallas import tpu as pltpu
from jax.experimental.pallas import tpu_sc as plsc

N = 10000
E = 160000
D = 256
DH = 128
NSC = 16
N_PAD = 10240
R = N_PAD // NSC
EPT = E // NSC
B = 80
NCHUNK = EPT // B

BM = 400


def _mm_y_body(x_ref, wn_ref, ya_ref, yb_ref):
    y = jnp.dot(x_ref[...], wn_ref[...], preferred_element_type=jnp.float32)
    ya_ref[...] = y[:, :DH]
    yb_ref[...] = y[:, DH:]


def _mm_y(x, wn):
    return pl.pallas_call(
        _mm_y_body,
        grid=(N // BM,),
        in_specs=[
            pl.BlockSpec((BM, D), lambda i: (i, 0)),
            pl.BlockSpec((D, D), lambda i: (0, 0)),
        ],
        out_specs=[
            pl.BlockSpec((BM, DH), lambda i: (i, 0)),
            pl.BlockSpec((BM, DH), lambda i: (i, 0)),
        ],
        out_shape=[
            jax.ShapeDtypeStruct((N_PAD, DH), jnp.float32),
            jax.ShapeDtypeStruct((N_PAD, DH), jnp.float32),
        ],
    )(x, wn)


def _mm_s_body(x_ref, ws_ref, b_ref, s_ref):
    s_ref[...] = jnp.dot(x_ref[...], ws_ref[...], preferred_element_type=jnp.float32) + b_ref[...]


def _mm_s(x, ws, b):
    return pl.pallas_call(
        _mm_s_body,
        grid=(N // BM,),
        in_specs=[
            pl.BlockSpec((BM, D), lambda i: (i, 0)),
            pl.BlockSpec((D, D), lambda i: (0, 0)),
            pl.BlockSpec((1, D), lambda i: (0, 0)),
        ],
        out_specs=pl.BlockSpec((BM, D), lambda i: (i, 0)),
        out_shape=jax.ShapeDtypeStruct((N_PAD, D), jnp.float32),
    )(x, ws, b.reshape(1, D))


def _h_of(s1_ref, aa_ref, ab_ref, c0_ref, c1_ref):
    inv = 1.0 / jnp.maximum(c0_ref[...] + c1_ref[...], 1.0)
    agg = jnp.concatenate([aa_ref[...], ab_ref[...]], axis=1) * inv
    return jnp.maximum(s1_ref[...] + agg, 0.0)


def _mid_y_body(s1_ref, aa_ref, ab_ref, c0_ref, c1_ref, wn_ref, ya_ref, yb_ref):
    h = _h_of(s1_ref, aa_ref, ab_ref, c0_ref, c1_ref)
    y = jnp.dot(h.astype(jnp.bfloat16), wn_ref[...].astype(jnp.bfloat16),
                preferred_element_type=jnp.float32)
    ya_ref[...] = y[:, :DH]
    yb_ref[...] = y[:, DH:]


def _mid_s_body(s1_ref, aa_ref, ab_ref, c0_ref, c1_ref, ws_ref, b_ref, s2_ref):
    h = _h_of(s1_ref, aa_ref, ab_ref, c0_ref, c1_ref)
    s2_ref[...] = jnp.dot(h, ws_ref[...], preferred_element_type=jnp.float32) + b_ref[...]


_MID_IN = [
    pl.BlockSpec((BM, D), lambda i: (i, 0)),
    pl.BlockSpec((BM, DH), lambda i: (i, 0)),
    pl.BlockSpec((BM, DH), lambda i: (i, 0)),
    pl.BlockSpec((BM, 1), lambda i: (i, 0)),
    pl.BlockSpec((BM, 1), lambda i: (i, 0)),
    pl.BlockSpec((D, D), lambda i: (0, 0)),
]


def _mid_y(s1, aa, ab, c0, c1, wn):
    return pl.pallas_call(
        _mid_y_body,
        grid=(N // BM,),
        in_specs=_MID_IN,
        out_specs=[
            pl.BlockSpec((BM, DH), lambda i: (i, 0)),
            pl.BlockSpec((BM, DH), lambda i: (i, 0)),
        ],
        out_shape=[
            jax.ShapeDtypeStruct((N_PAD, DH), jnp.float32),
            jax.ShapeDtypeStruct((N_PAD, DH), jnp.float32),
        ],
    )(s1, aa, ab, c0, c1, wn)


def _mid_s(s1, aa, ab, c0, c1, ws, b):
    return pl.pallas_call(
        _mid_s_body,
        grid=(N // BM,),
        in_specs=_MID_IN + [pl.BlockSpec((1, D), lambda i: (0, 0))],
        out_specs=pl.BlockSpec((BM, D), lambda i: (i, 0)),
        out_shape=jax.ShapeDtypeStruct((N_PAD, D), jnp.float32),
    )(s1, aa, ab, c0, c1, ws, b.reshape(1, D))


def _fin_body(s2_ref, aa_ref, ab_ref, c0_ref, c1_ref, o_ref):
    inv = 1.0 / jnp.maximum(c0_ref[...] + c1_ref[...], 1.0)
    o_ref[...] = s2_ref[...] + jnp.concatenate([aa_ref[...], ab_ref[...]], axis=1) * inv


def _fin(s2, aa, ab, c0, c1):
    return pl.pallas_call(
        _fin_body,
        grid=(N // BM,),
        in_specs=[
            pl.BlockSpec((BM, D), lambda i: (i, 0)),
            pl.BlockSpec((BM, DH), lambda i: (i, 0)),
            pl.BlockSpec((BM, DH), lambda i: (i, 0)),
            pl.BlockSpec((BM, 1), lambda i: (i, 0)),
            pl.BlockSpec((BM, 1), lambda i: (i, 0)),
        ],
        out_specs=pl.BlockSpec((BM, D), lambda i: (i, 0)),
        out_shape=jax.ShapeDtypeStruct((N, D), jnp.float32),
    )(s2, aa, ab, c0, c1)


ECNT = E // 32


@functools.lru_cache(maxsize=None)
def _make_segsum():
    mesh = plsc.VectorSubcoreMesh(
        core_axis_name="c", subcore_axis_name="s", num_cores=2, num_subcores=NSC)

    DP = 4
    IP = 6

    out_type = [
        jax.ShapeDtypeStruct((N_PAD, DH), jnp.float32),
        jax.ShapeDtypeStruct((N_PAD, DH), jnp.float32),
    ]
    scratch = [
        pltpu.VMEM((IP, B), jnp.int32),
        pltpu.VMEM((IP, B), jnp.int32),
        pltpu.VMEM((DP, B, DH), jnp.float32),
        pltpu.VMEM_SHARED((N_PAD, DH), jnp.float32),
        pltpu.SemaphoreType.DMA,
        pltpu.SemaphoreType.DMA,
        pltpu.SemaphoreType.DMA,
        pltpu.SemaphoreType.DMA,
    ]

    def body(ya, yb, ei, agg_a, agg_b,
             sidx, didx, rows, accum, sisem, disem, gsem, ssem):
        c = lax.axis_index("c")
        s = lax.axis_index("s")
        row0 = s * R
        ebase = s * EPT

        def _zr(r, carry):
            for k in range(DH // 16):
                rows[0, r, pl.ds(k * 16, 16)] = jnp.zeros((16,), jnp.float32)
            return carry
        lax.fori_loop(0, B, _zr, 0)
        for q in range(R // B):
            pltpu.sync_copy(rows.at[0], accum.at[pl.ds(row0 + q * B, B)])
        plsc.subcore_barrier()

        def load_idx(i):
            buf = lax.rem(i, IP)
            off = ebase + i * B
            pltpu.async_copy(ei.at[pl.ds(off, B)], sidx.at[buf], sisem)
            pltpu.async_copy(ei.at[pl.ds(E + off, B)], didx.at[buf], disem)

        def gather(i):
            buf = lax.rem(i, DP)

            @pl.when(c == 0)
            def _():
                pltpu.async_copy(ya.at[sidx.at[lax.rem(i, IP)]], rows.at[buf], gsem)

            @pl.when(c == 1)
            def _():
                pltpu.async_copy(yb.at[sidx.at[lax.rem(i, IP)]], rows.at[buf], gsem)

        def wait_idx(sem):
            pltpu.make_async_copy(ei.at[pl.ds(0, B)], sidx.at[0], sem).wait()

        def wait_gather():
            pltpu.make_async_copy(ya.at[sidx.at[0]], rows.at[0], gsem).wait()

        def wait_scatter():
            pltpu.make_async_copy(rows.at[0], accum.at[didx.at[0]], ssem).wait()

        for k in range(4):
            load_idx(k)
        wait_idx(sisem)
        gather(0)
        wait_idx(sisem)
        gather(1)

        def step(i, carry):
            @pl.when(i >= 2)
            def _():
                wait_scatter()

            @pl.when(i + 4 < NCHUNK)
            def _():
                load_idx(i + 4)

            @pl.when(i + 2 < NCHUNK)
            def _():
                wait_idx(sisem)
                gather(i + 2)

            wait_gather()
            wait_idx(disem)
            pltpu.async_copy(rows.at[lax.rem(i, DP)],
                             accum.at[didx.at[lax.rem(i, IP)]], ssem, add=True)
            return carry

        lax.fori_loop(0, NCHUNK, step, 0)
        wait_scatter()
        wait_scatter()
        plsc.subcore_barrier()

        @pl.when(c == 0)
        def _():
            pltpu.sync_copy(accum.at[pl.ds(row0, R)], agg_a.at[pl.ds(row0, R)])

        @pl.when(c == 1)
        def _():
            pltpu.sync_copy(accum.at[pl.ds(row0, R)], agg_b.at[pl.ds(row0, R)])

    return pl.kernel(
        body, out_type=out_type, mesh=mesh, scratch_types=scratch,
        compiler_params=pltpu.CompilerParams(needs_layout_passes=False))


@functools.lru_cache(maxsize=None)
def _make_cnt():
    mesh = plsc.VectorSubcoreMesh(
        core_axis_name="c", subcore_axis_name="s", num_cores=2, num_subcores=NSC)

    out_type = [
        jax.ShapeDtypeStruct((N_PAD,), jnp.float32),
        jax.ShapeDtypeStruct((N_PAD,), jnp.float32),
    ]
    scratch = [
        pltpu.VMEM((ECNT,), jnp.int32),
        pltpu.VMEM((N_PAD,), jnp.float32),
        pltpu.VMEM((NSC, R), jnp.float32),
        pltpu.VMEM((R,), jnp.float32),
        pltpu.VMEM_SHARED((NSC, N_PAD), jnp.float32),
    ]

    def body(ei, c0_out, c1_out, didx, cntp, cred, cout, cnt_all):
        c = lax.axis_index("c")
        s = lax.axis_index("s")
        w = c * NSC + s
        row0 = s * R

        pltpu.sync_copy(ei.at[pl.ds(E + w * ECNT, ECNT)], didx)

        def _zc(i, carry):
            cntp[pl.ds(i * 16, 16)] = jnp.zeros((16,), jnp.float32)
            return carry
        lax.fori_loop(0, N_PAD // 16, _zc, 0)

        ones16 = jnp.ones((16,), jnp.float32)

        def _cc(i, carry):
            d16 = didx[pl.ds(i * 16, 16)]
            plsc.addupdate_scatter(cntp, [d16], ones16)
            return carry
        lax.fori_loop(0, ECNT // 16, _cc, 0)
        d16 = didx[pl.ds(ECNT - 16, 16)]
        lanes = lax.broadcasted_iota(jnp.int32, (16,), 0)
        plsc.addupdate_scatter(cntp, [d16], ones16, mask=lanes >= 8)

        pltpu.sync_copy(cntp, cnt_all.at[s])
        plsc.subcore_barrier()
        pltpu.sync_copy(cnt_all.at[:, pl.ds(row0, R)], cred)

        def red(j, carry):
            acc = jnp.zeros((16,), jnp.float32)
            for r in range(NSC):
                acc = acc + cred[r, pl.ds(j * 16, 16)]
            cout[pl.ds(j * 16, 16)] = acc
            return carry
        lax.fori_loop(0, R // 16, red, 0)

        @pl.when(c == 0)
        def _():
            pltpu.sync_copy(cout, c0_out.at[pl.ds(row0, R)])

        @pl.when(c == 1)
        def _():
            pltpu.sync_copy(cout, c1_out.at[pl.ds(row0, R)])

    return pl.kernel(
        body, out_type=out_type, mesh=mesh, scratch_types=scratch,
        compiler_params=pltpu.CompilerParams(needs_layout_passes=False))


@jax.jit
def kernel(x, edge_index, W1_self, W1_neigh, b1, W2_self, W2_neigh, b2):
    ei = edge_index.reshape(2 * E)
    cnt0, cnt1 = _make_cnt()(ei)
    c0 = cnt0.reshape(N_PAD, 1)
    c1 = cnt1.reshape(N_PAD, 1)
    y1a, y1b = _mm_y(x, W1_neigh)
    agg_a, agg_b = _make_segsum()(y1a, y1b, ei)
    s1 = _mm_s(x, W1_self, b1)
    y2a, y2b = _mid_y(s1, agg_a, agg_b, c0, c1, W2_neigh)
    agg_a2, agg_b2 = _make_segsum()(y2a, y2b, ei)
    s2 = _mid_s(s1, agg_a, agg_b, c0, c1, W2_self, b2)
    return _fin(s2, agg_a2, agg_b2, c0, c1)
